# baseline (speedup 1.0000x reference)
import jax
import jax.numpy as jnp
import numpy as np
from jax.experimental import pallas as pl
from jax.experimental.pallas import tpu as pltpu

_B, _T, _N, _D, _H, _C, _L = 8, 8, 128, 128, 128, 8, 4
_G = _B * _T
_R = _G * _N
_K_LOW = 734002
_ONE_BITS = 0x3F800000
_INV_R = np.float32(1.0 / _R)
_INV_N = np.float32(1.0 / _N)
_SQRT_C = np.float32(np.sqrt(128.0))


def _fwd(v_ref, a_ref, wi_ref, bi_ref,
         w1_ref, b1_ref, g1_ref, be1_ref,
         w2_ref, b2_ref, g2_ref, be2_ref,
         wq_ref, bq_ref, wk_ref, bk_ref,
         wl_ref, bl_ref, eps_ref,
         out_ref, h_ref, ha_ref, at_ref):
    av = a_ref[...]
    bits = jax.lax.bitcast_convert_type(av, jnp.int32)

    def count_le(t):
        return jnp.sum((bits <= t).astype(jnp.int32))

    def bs_body(_, state):
        lo, hi, chi = state
        d = hi - lo
        q = d >> 2
        m1 = lo + q
        m2 = lo + (d >> 1)
        m3 = m2 + q
        c1 = count_le(m1)
        c2 = count_le(m2)
        c3 = count_le(m3)
        p1 = c1 >= _K_LOW + 1
        p2 = c2 >= _K_LOW + 1
        p3 = c3 >= _K_LOW + 1
        lo_n = jnp.where(p1, lo, jnp.where(p2, m1, jnp.where(p3, m2, m3)))
        hi_n = jnp.where(p1, m1, jnp.where(p2, m2, jnp.where(p3, m3, hi)))
        chi_n = jnp.where(p1, c1, jnp.where(p2, c2, jnp.where(p3, c3, chi)))
        return lo_n, hi_n, chi_n

    lo0 = jnp.int32(-1)
    hi0 = jnp.int32(_ONE_BITS)
    _, s_low, c_low = jax.lax.fori_loop(
        0, 17, bs_body, (lo0, hi0, jnp.int32(_G * _N * _N)))
    nxt = jnp.min(jnp.where(bits > s_low, bits, jnp.int32(0x7FFFFFFF)))
    s_high = jnp.where(c_low >= _K_LOW + 2, s_low, nxt)
    fl = jax.lax.bitcast_convert_type(jnp.full((8, 128), s_low, jnp.int32),
                                      jnp.float32)
    fh = jax.lax.bitcast_convert_type(jnp.full((8, 128), s_high, jnp.int32),
                                      jnp.float32)
    thr = jnp.max(fl * 0.5 + fh * 0.5)
    at_ref[...] = (av > thr).astype(jnp.float32)

    h_ref[...] = (jnp.dot(v_ref[...], wi_ref[...],
                          preferred_element_type=jnp.float32) + bi_ref[...])

    logit = jnp.zeros((_B, _C), jnp.float32)
    for l in range(_L):
        eps_row = eps_ref[l]

        def spmm_body(g, carry):
            r0 = g * _N
            hg = h_ref[pl.ds(r0, _N), :]
            ha_ref[pl.ds(r0, _N), :] = (
                jnp.dot(at_ref[g], hg, preferred_element_type=jnp.float32)
                + eps_row * hg)
            return carry

        jax.lax.fori_loop(0, _G, spmm_body, 0, unroll=16)

        z = (jnp.dot(ha_ref[...], w1_ref[l],
                     preferred_element_type=jnp.float32) + b1_ref[l])
        m = jnp.sum(z, axis=0, keepdims=True) * _INV_R
        var = jnp.sum(z * z, axis=0, keepdims=True) * _INV_R - m * m
        sc1 = g1_ref[l] / jnp.sqrt(var + 1e-5)
        sh1 = be1_ref[l] - m * sc1
        z = jnp.maximum(z * sc1 + sh1, 0.0)

        z = (jnp.dot(z, w2_ref[l],
                     preferred_element_type=jnp.float32) + b2_ref[l])
        m2 = jnp.sum(z, axis=0, keepdims=True) * _INV_R
        var2 = jnp.sum(z * z, axis=0, keepdims=True) * _INV_R - m2 * m2
        sc2 = g2_ref[l] / jnp.sqrt(var2 + 1e-5)
        sh2 = be2_ref[l] - m2 * sc2
        h = jnp.maximum(z * sc2 + sh2, 0.0)
        h_ref[...] = h

        wq = wq_ref[l]
        bq_row = bq_ref[l]
        wk = wk_ref[l]
        bk_row = bk_ref[l]

        def ro_body(g, carry):
            hg = h_ref[pl.ds(g * _N, _N), :]
            xm = jnp.sum(hg, axis=0, keepdims=True) * _INV_N
            xq = jnp.dot(xm, wq, preferred_element_type=jnp.float32) + bq_row
            xk = jnp.dot(hg, wk, preferred_element_type=jnp.float32) + bk_row
            al = jnp.sum(xk * xq, axis=1, keepdims=True)
            attn = jax.nn.sigmoid(al / _SQRT_C)
            ha_ref[pl.ds(g, 1), :] = (
                jnp.sum(hg * attn, axis=0, keepdims=True) * _INV_N)
            return carry

        jax.lax.fori_loop(0, _G, ro_body, 0, unroll=8)
        lat = jnp.sum(ha_ref[0:_G, :].reshape(_B, _T, _H), axis=1)
        logit = (logit
                 + jnp.dot(lat, wl_ref[l], preferred_element_type=jnp.float32)
                 + bl_ref[l])

    out_ref[...] = logit


def kernel(v, a, params):
    lyr = params['layers']
    v2 = v.reshape(_R, _D)
    a3 = a.reshape(_G, _N, _N)
    wi = params['Wi']
    bi = params['bi'].reshape(1, _H)
    stk = lambda k: jnp.stack([p[k] for p in lyr])
    w1s, w2s, wqs, wks, wls = stk('W1'), stk('W2'), stk('Wq'), stk('Wk'), stk('Wl')
    b1s, g1s, be1s = stk('b1'), stk('g1'), stk('be1')
    b2s, g2s, be2s = stk('b2'), stk('g2'), stk('be2')
    bqs, bks, bls = stk('bq'), stk('bk'), stk('bl')
    epss = jnp.stack([jnp.broadcast_to(p['eps'][0, 0], (_H,)) for p in lyr])

    return pl.pallas_call(
        _fwd,
        out_shape=jax.ShapeDtypeStruct((_B, _C), jnp.float32),
        in_specs=[pl.BlockSpec(memory_space=pltpu.VMEM)] * 19,
        out_specs=pl.BlockSpec(memory_space=pltpu.VMEM),
        scratch_shapes=[
            pltpu.VMEM((_R, _H), jnp.float32),
            pltpu.VMEM((_R, _H), jnp.float32),
            pltpu.VMEM((_G, _N, _N), jnp.float32),
        ],
    )(v2, a3, wi, bi, w1s, b1s, g1s, be1s, w2s, b2s, g2s, be2s,
      wqs, bqs, wks, bks, wls, bls, epss)


# speedup vs baseline: 1.1218x; 1.1218x over previous
import jax
import jax.numpy as jnp
import numpy as np
from jax.experimental import pallas as pl
from jax.experimental.pallas import tpu as pltpu

_B, _T, _N, _D, _H, _C, _L = 8, 8, 128, 128, 128, 8, 4
_G = _B * _T
_R = _G * _N
_K_LOW = 734002
_ONE_BITS = 0x3F800000
_INV_R = np.float32(1.0 / _R)
_INV_N = np.float32(1.0 / _N)
_SQRT_C = np.float32(np.sqrt(128.0))


def _fwd(v_ref, a_ref, wi_ref, bi_ref,
         w1_ref, b1_ref, g1_ref, be1_ref,
         w2_ref, b2_ref, g2_ref, be2_ref,
         wq_ref, bq_ref, wk_ref, bk_ref,
         wl_ref, bl_ref, eps_ref,
         out_ref, h_ref, ha_ref, at_ref):
    av = a_ref[...]
    bits = jax.lax.bitcast_convert_type(av, jnp.int32)

    def count_le(t):
        return jnp.sum((bits <= t).astype(jnp.int32))

    def bs_body(_, state):
        lo, hi, chi = state
        d = hi - lo
        q = d >> 2
        m1 = lo + q
        m2 = lo + (d >> 1)
        m3 = m2 + q
        c1 = count_le(m1)
        c2 = count_le(m2)
        c3 = count_le(m3)
        p1 = c1 >= _K_LOW + 1
        p2 = c2 >= _K_LOW + 1
        p3 = c3 >= _K_LOW + 1
        lo_n = jnp.where(p1, lo, jnp.where(p2, m1, jnp.where(p3, m2, m3)))
        hi_n = jnp.where(p1, m1, jnp.where(p2, m2, jnp.where(p3, m3, hi)))
        chi_n = jnp.where(p1, c1, jnp.where(p2, c2, jnp.where(p3, c3, chi)))
        return lo_n, hi_n, chi_n

    lo0 = jnp.int32(-1)
    hi0 = jnp.int32(0x3FFFFFFF)
    _, s_low, c_low = jax.lax.fori_loop(
        0, 15, bs_body, (lo0, hi0, jnp.int32(_G * _N * _N)))
    nxt = jnp.min(jnp.where(bits > s_low, bits, jnp.int32(0x7FFFFFFF)))
    s_high = jnp.where(c_low >= _K_LOW + 2, s_low, nxt)
    fl = jax.lax.bitcast_convert_type(jnp.full((8, 128), s_low, jnp.int32),
                                      jnp.float32)
    fh = jax.lax.bitcast_convert_type(jnp.full((8, 128), s_high, jnp.int32),
                                      jnp.float32)
    thr = jnp.max(fl * 0.5 + fh * 0.5)
    at_ref[...] = (av > thr).astype(jnp.float32)

    h_ref[...] = (jnp.dot(v_ref[...], wi_ref[...],
                          preferred_element_type=jnp.float32) + bi_ref[...])

    logit = jnp.zeros((_B, _C), jnp.float32)
    for l in range(_L):
        eps_row = eps_ref[l]

        def spmm_body(g, carry):
            r0 = g * _N
            hg = h_ref[pl.ds(r0, _N), :]
            ha_ref[pl.ds(r0, _N), :] = (
                jnp.dot(at_ref[g], hg, preferred_element_type=jnp.float32)
                + eps_row * hg)
            return carry

        jax.lax.fori_loop(0, _G, spmm_body, 0, unroll=16)

        z = (jnp.dot(ha_ref[...], w1_ref[l],
                     preferred_element_type=jnp.float32) + b1_ref[l])
        m = jnp.sum(z, axis=0, keepdims=True) * _INV_R
        var = jnp.sum(z * z, axis=0, keepdims=True) * _INV_R - m * m
        sc1 = g1_ref[l] / jnp.sqrt(var + 1e-5)
        sh1 = be1_ref[l] - m * sc1
        z = jnp.maximum(z * sc1 + sh1, 0.0)

        z = (jnp.dot(z, w2_ref[l],
                     preferred_element_type=jnp.float32) + b2_ref[l])
        m2 = jnp.sum(z, axis=0, keepdims=True) * _INV_R
        var2 = jnp.sum(z * z, axis=0, keepdims=True) * _INV_R - m2 * m2
        sc2 = g2_ref[l] / jnp.sqrt(var2 + 1e-5)
        sh2 = be2_ref[l] - m2 * sc2
        h = jnp.maximum(z * sc2 + sh2, 0.0)
        h_ref[...] = h

        h3 = h.reshape(_G, _N, _H)
        xm = jnp.sum(h3, axis=1) * _INV_N
        xq = (jnp.dot(xm, wq_ref[l],
                      preferred_element_type=jnp.float32) + bq_ref[l])
        xk = (jnp.dot(h, wk_ref[l],
                      preferred_element_type=jnp.float32) + bk_ref[l])
        al = jnp.sum(xk.reshape(_G, _N, _H) * xq[:, None, :], axis=2)
        attn = jax.nn.sigmoid(al / _SQRT_C)
        hr = jnp.sum(h3 * attn[:, :, None], axis=1) * _INV_N
        lat = jnp.sum(hr.reshape(_B, _T, _H), axis=1)
        logit = (logit
                 + jnp.dot(lat, wl_ref[l], preferred_element_type=jnp.float32)
                 + bl_ref[l])

    out_ref[...] = logit


def kernel(v, a, params):
    lyr = params['layers']
    v2 = v.reshape(_R, _D)
    a3 = a.reshape(_G, _N, _N)
    wi = params['Wi']
    bi = params['bi'].reshape(1, _H)
    stk = lambda k: jnp.stack([p[k] for p in lyr])
    w1s, w2s, wqs, wks, wls = stk('W1'), stk('W2'), stk('Wq'), stk('Wk'), stk('Wl')
    b1s, g1s, be1s = stk('b1'), stk('g1'), stk('be1')
    b2s, g2s, be2s = stk('b2'), stk('g2'), stk('be2')
    bqs, bks, bls = stk('bq'), stk('bk'), stk('bl')
    epss = jnp.stack([jnp.broadcast_to(p['eps'][0, 0], (_H,)) for p in lyr])

    return pl.pallas_call(
        _fwd,
        out_shape=jax.ShapeDtypeStruct((_B, _C), jnp.float32),
        in_specs=[pl.BlockSpec(memory_space=pltpu.VMEM)] * 19,
        out_specs=pl.BlockSpec(memory_space=pltpu.VMEM),
        scratch_shapes=[
            pltpu.VMEM((_R, _H), jnp.float32),
            pltpu.VMEM((_R, _H), jnp.float32),
            pltpu.VMEM((_G, _N, _N), jnp.float32),
        ],
    )(v2, a3, wi, bi, w1s, b1s, g1s, be1s, w2s, b2s, g2s, be2s,
      wqs, bqs, wks, bks, wls, bls, epss)


# speedup vs baseline: 1.1233x; 1.0013x over previous
import jax
import jax.numpy as jnp
import numpy as np
from jax.experimental import pallas as pl
from jax.experimental.pallas import tpu as pltpu

_B, _T, _N, _D, _H, _C, _L = 8, 8, 128, 128, 128, 8, 4
_G = _B * _T
_R = _G * _N
_K_LOW = 734002
_ONE_BITS = 0x3F800000
_INV_R = np.float32(1.0 / _R)
_INV_N = np.float32(1.0 / _N)
_SQRT_C = np.float32(np.sqrt(128.0))


def _fwd(v_ref, a_ref, wi_ref, bi_ref,
         w1_ref, b1_ref, g1_ref, be1_ref,
         w2_ref, b2_ref, g2_ref, be2_ref,
         wq_ref, bq_ref, wk_ref, bk_ref,
         wl_ref, bl_ref, eps_ref,
         out_ref, h_ref, ha_ref, at_ref):
    av = a_ref[...]
    bits = jax.lax.bitcast_convert_type(av, jnp.int32)

    def count_le(t):
        return jnp.sum((bits <= t).astype(jnp.int32))

    def bs_body(_, state):
        lo, hi, chi = state
        d = hi - lo
        q = d >> 2
        m1 = lo + q
        m2 = lo + (d >> 1)
        m3 = m2 + q
        c1 = count_le(m1)
        c2 = count_le(m2)
        c3 = count_le(m3)
        p1 = c1 >= _K_LOW + 1
        p2 = c2 >= _K_LOW + 1
        p3 = c3 >= _K_LOW + 1
        lo_n = jnp.where(p1, lo, jnp.where(p2, m1, jnp.where(p3, m2, m3)))
        hi_n = jnp.where(p1, m1, jnp.where(p2, m2, jnp.where(p3, m3, hi)))
        chi_n = jnp.where(p1, c1, jnp.where(p2, c2, jnp.where(p3, c3, chi)))
        return lo_n, hi_n, chi_n

    lo0 = jnp.int32(-1)
    hi0 = jnp.int32(0x3FFFFFFF)
    _, s_low, c_low = jax.lax.fori_loop(
        0, 15, bs_body, (lo0, hi0, jnp.int32(_G * _N * _N)))
    nxt = jnp.min(jnp.where(bits > s_low, bits, jnp.int32(0x7FFFFFFF)))
    s_high = jnp.where(c_low >= _K_LOW + 2, s_low, nxt)
    fl = jax.lax.bitcast_convert_type(jnp.full((8, 128), s_low, jnp.int32),
                                      jnp.float32)
    fh = jax.lax.bitcast_convert_type(jnp.full((8, 128), s_high, jnp.int32),
                                      jnp.float32)
    thr = jnp.max(fl * 0.5 + fh * 0.5)

    h_ref[...] = (jnp.dot(v_ref[...], wi_ref[...],
                          preferred_element_type=jnp.float32) + bi_ref[...])

    logit = jnp.zeros((_B, _C), jnp.float32)
    for l in range(_L):
        eps_row = eps_ref[l]

        if l == 0:
            def spmm_body(g, carry):
                r0 = g * _N
                hg = h_ref[pl.ds(r0, _N), :]
                ag = (a_ref[g] > thr).astype(jnp.float32)
                at_ref[g] = ag
                ha_ref[pl.ds(r0, _N), :] = (
                    jnp.dot(ag, hg, preferred_element_type=jnp.float32)
                    + eps_row * hg)
                return carry
        else:
            def spmm_body(g, carry):
                r0 = g * _N
                hg = h_ref[pl.ds(r0, _N), :]
                ha_ref[pl.ds(r0, _N), :] = (
                    jnp.dot(at_ref[g], hg, preferred_element_type=jnp.float32)
                    + eps_row * hg)
                return carry

        jax.lax.fori_loop(0, _G, spmm_body, 0, unroll=16)

        z = (jnp.dot(ha_ref[...], w1_ref[l],
                     preferred_element_type=jnp.float32) + b1_ref[l])
        m = jnp.sum(z, axis=0, keepdims=True) * _INV_R
        var = jnp.sum(z * z, axis=0, keepdims=True) * _INV_R - m * m
        sc1 = g1_ref[l] / jnp.sqrt(var + 1e-5)
        sh1 = be1_ref[l] - m * sc1
        z = jnp.maximum(z * sc1 + sh1, 0.0)

        z = (jnp.dot(z, w2_ref[l],
                     preferred_element_type=jnp.float32) + b2_ref[l])
        m2 = jnp.sum(z, axis=0, keepdims=True) * _INV_R
        var2 = jnp.sum(z * z, axis=0, keepdims=True) * _INV_R - m2 * m2
        sc2 = g2_ref[l] / jnp.sqrt(var2 + 1e-5)
        sh2 = be2_ref[l] - m2 * sc2
        h = jnp.maximum(z * sc2 + sh2, 0.0)
        h_ref[...] = h

        h3 = h.reshape(_G, _N, _H)
        xm = jnp.sum(h3, axis=1) * _INV_N
        xq = (jnp.dot(xm, wq_ref[l],
                      preferred_element_type=jnp.float32) + bq_ref[l])
        xk = (jnp.dot(h, wk_ref[l],
                      preferred_element_type=jnp.float32) + bk_ref[l])
        al = jnp.sum(xk.reshape(_G, _N, _H) * xq[:, None, :], axis=2)
        attn = jax.nn.sigmoid(al / _SQRT_C)
        hr = jnp.sum(h3 * attn[:, :, None], axis=1) * _INV_N
        lat = jnp.sum(hr.reshape(_B, _T, _H), axis=1)
        logit = (logit
                 + jnp.dot(lat, wl_ref[l], preferred_element_type=jnp.float32)
                 + bl_ref[l])

    out_ref[...] = logit


def kernel(v, a, params):
    lyr = params['layers']
    v2 = v.reshape(_R, _D)
    a3 = a.reshape(_G, _N, _N)
    wi = params['Wi']
    bi = params['bi'].reshape(1, _H)
    stk = lambda k: jnp.stack([p[k] for p in lyr])
    w1s, w2s, wqs, wks, wls = stk('W1'), stk('W2'), stk('Wq'), stk('Wk'), stk('Wl')
    b1s, g1s, be1s = stk('b1'), stk('g1'), stk('be1')
    b2s, g2s, be2s = stk('b2'), stk('g2'), stk('be2')
    bqs, bks, bls = stk('bq'), stk('bk'), stk('bl')
    epss = jnp.stack([jnp.broadcast_to(p['eps'][0, 0], (_H,)) for p in lyr])

    return pl.pallas_call(
        _fwd,
        out_shape=jax.ShapeDtypeStruct((_B, _C), jnp.float32),
        in_specs=[pl.BlockSpec(memory_space=pltpu.VMEM)] * 19,
        out_specs=pl.BlockSpec(memory_space=pltpu.VMEM),
        scratch_shapes=[
            pltpu.VMEM((_R, _H), jnp.float32),
            pltpu.VMEM((_R, _H), jnp.float32),
            pltpu.VMEM((_G, _N, _N), jnp.float32),
        ],
    )(v2, a3, wi, bi, w1s, b1s, g1s, be1s, w2s, b2s, g2s, be2s,
      wqs, bqs, wks, bks, wls, bls, epss)
